# Initial kernel scaffold; baseline (speedup 1.0000x reference)
import jax
import jax.numpy as jnp
from jax.experimental import pallas as pl


def kernel(x, Wr, br, W1, b1, W2, b2):
    raise NotImplementedError("write your pallas kernel here")


# speedup vs baseline: 1.0245x; 1.0245x over previous
import functools

import jax
import jax.numpy as jnp
from jax.experimental import pallas as pl
from jax.experimental.pallas import tpu as pltpu

TOPK = 2


def _router_body(x_ref, wr_ref, br_ref, gate_ref):
    xb = x_ref[...]
    logits = jnp.dot(xb, wr_ref[...], preferred_element_type=jnp.float32)
    logits = logits + br_ref[...]
    bt, e = logits.shape
    iota_e = jax.lax.broadcasted_iota(jnp.int32, (bt, e), 1)
    m1 = jnp.max(logits, axis=-1, keepdims=True)
    i1 = jnp.min(jnp.where(logits == m1, iota_e, e), axis=-1, keepdims=True)
    l2 = jnp.where(iota_e == i1, -jnp.inf, logits)
    m2 = jnp.max(l2, axis=-1, keepdims=True)
    i2 = jnp.min(jnp.where(l2 == m2, iota_e, e), axis=-1, keepdims=True)
    g1 = 1.0 / (1.0 + jnp.exp(m2 - m1))
    g2 = 1.0 - g1
    gate = jnp.where(iota_e == i1, g1, jnp.where(iota_e == i2, g2, 0.0))
    gate_ref[...] = gate


def _router(x_flat, Wr, br, bt):
    T, D = x_flat.shape
    E = Wr.shape[1]
    return pl.pallas_call(
        _router_body,
        grid=(T // bt,),
        in_specs=[
            pl.BlockSpec((bt, D), lambda t: (t, 0)),
            pl.BlockSpec((D, E), lambda t: (0, 0)),
            pl.BlockSpec((1, E), lambda t: (0, 0)),
        ],
        out_specs=pl.BlockSpec((bt, E), lambda t: (t, 0)),
        out_shape=jax.ShapeDtypeStruct((T, E), jnp.float32),
    )(x_flat, Wr, br.reshape(1, E))


def _ffn_body(x_ref, w1_ref, b1_ref, w2_ref, b2_ref, g_ref, out_ref, acc_ref,
              *, n_e, n_hb):
    e = pl.program_id(1)
    hb = pl.program_id(2)

    @pl.when(jnp.logical_and(e == 0, hb == 0))
    def _():
        acc_ref[...] = jnp.zeros_like(acc_ref)

    h = jnp.dot(x_ref[...], w1_ref[0], preferred_element_type=jnp.float32)
    h = h + b1_ref[0]
    h = h * (1.0 / (1.0 + jnp.exp(-h)))
    part = jnp.dot(h, w2_ref[0], preferred_element_type=jnp.float32)
    gfull = g_ref[...]
    col = jax.lax.broadcasted_iota(jnp.int32, gfull.shape, 1)
    g = jnp.sum(jnp.where(col == e, gfull, 0.0), axis=1, keepdims=True)
    contrib = part * g

    @pl.when(hb == 0)
    def _():
        acc_ref[...] += b2_ref[0] * g

    acc_ref[...] += contrib

    @pl.when(jnp.logical_and(e == n_e - 1, hb == n_hb - 1))
    def _():
        out_ref[...] = acc_ref[...]


def _dense_ffn(x_flat, W1, b1, W2, b2, gate, bt, bh):
    T, D = x_flat.shape
    E, _, H = W1.shape
    n_tb, n_hb = T // bt, H // bh
    return pl.pallas_call(
        functools.partial(_ffn_body, n_e=E, n_hb=n_hb),
        grid=(n_tb, E, n_hb),
        in_specs=[
            pl.BlockSpec((bt, D), lambda t, e, h: (t, 0)),
            pl.BlockSpec((1, D, bh), lambda t, e, h: (e, 0, h)),
            pl.BlockSpec((1, 1, bh), lambda t, e, h: (e, 0, h)),
            pl.BlockSpec((1, bh, D), lambda t, e, h: (e, h, 0)),
            pl.BlockSpec((1, 1, D), lambda t, e, h: (e, 0, 0)),
            pl.BlockSpec((bt, E), lambda t, e, h: (t, 0)),
        ],
        out_specs=pl.BlockSpec((bt, D), lambda t, e, h: (t, 0)),
        out_shape=jax.ShapeDtypeStruct((T, D), jnp.float32),
        scratch_shapes=[pltpu.VMEM((bt, D), jnp.float32)],
    )(x_flat, W1, b1.reshape(E, 1, H), W2, b2.reshape(E, 1, D), gate)


@jax.jit
def kernel(x, Wr, br, W1, b1, W2, b2):
    B, S, D = x.shape
    x_flat = x.reshape(-1, D)
    T = x_flat.shape[0]
    bt = min(512, T)
    gate = _router(x_flat, Wr, br, bt)
    out = _dense_ffn(x_flat, W1, b1, W2, b2, gate, bt, min(1024, W1.shape[2]))
    return out.reshape(x.shape)
